# Initial kernel scaffold; baseline (speedup 1.0000x reference)
import jax
import jax.numpy as jnp
from jax.experimental import pallas as pl


def kernel(x, edge_index, edge_weight, weight1, w_ih1, w_hh1, b_ih1, b_hh1, weight2, w_ih2, w_hh2, b_ih2, b_hh2):
    raise NotImplementedError("write your pallas kernel here")


# speedup vs baseline: 15.2699x; 15.2699x over previous
import functools

import jax
import jax.numpy as jnp
from jax import lax
from jax.experimental import pallas as pl
from jax.experimental.pallas import tpu as pltpu
from jax.experimental.pallas import tpu_sc as plsc

N = 50000
E = 1600000
F = 16

NCORES = 2
NSUB = 16
NW = NCORES * NSUB

NP = 51200
EP = 1638400

CHUNK = 128
SUPER = 2048
NCHUNK = SUPER // CHUNK
EDGES_PER_W = EP // NW
NSUPER = EDGES_PER_W // SUPER
IDXROWS = EP // CHUNK
ROWS_PER_W = EDGES_PER_W // CHUNK
AGG_ROWS_PER_TILE = NP // NSUB

PACK = 8
NPACK = NP * F // 128
GBLK = 1280


def _sc_agg_body(m_hbm, src_hbm, dst_hbm, ew_hbm, out_hbm,
                 agg_sh, zbuf, sidx, didx, wbuf, rows, gsem):
    c = lax.axis_index("c")
    s = lax.axis_index("s")
    w = c * NSUB + s

    def zfill(j, carry):
        zbuf[j] = jnp.zeros((F,), jnp.float32)
        return carry
    lax.fori_loop(0, CHUNK, zfill, 0, unroll=8)

    aggbase = s * AGG_ROWS_PER_TILE

    def zcopy(j, carry):
        pltpu.sync_copy(zbuf, agg_sh.at[pl.ds(aggbase + j * CHUNK, CHUNK)])
        return carry
    lax.fori_loop(0, AGG_ROWS_PER_TILE // CHUNK, zcopy, 0)

    plsc.subcore_barrier()

    ebase = w * EDGES_PER_W
    rbase = w * ROWS_PER_W

    def do_super(sc_i, carry):
        r0 = rbase + sc_i * NCHUNK
        e0 = ebase + sc_i * SUPER
        pltpu.sync_copy(src_hbm.at[pl.ds(r0, NCHUNK)], sidx)
        pltpu.sync_copy(dst_hbm.at[pl.ds(r0, NCHUNK)], didx)
        pltpu.sync_copy(ew_hbm.at[pl.ds(e0, SUPER)], wbuf)
        handles = []
        for j in range(NCHUNK):
            handles.append(pltpu.async_copy(
                m_hbm.at[sidx.at[j]],
                rows.at[pl.ds(j * CHUNK, CHUNK)],
                gsem))
        for h in handles:
            h.wait()

        def mul(g_i, carry2):
            base = g_i * F
            wv = wbuf[pl.ds(base, F)]
            for j in range(F):
                rows[base + j] = rows[base + j] * wv[j]
            return carry2
        lax.fori_loop(0, SUPER // F, mul, 0)

        for j in range(NCHUNK):
            pltpu.sync_copy(rows.at[pl.ds(j * CHUNK, CHUNK)],
                            agg_sh.at[didx.at[j]],
                            add=True)
        return carry
    lax.fori_loop(0, NSUPER, do_super, 0)

    plsc.subcore_barrier()

    outbase = c * NP + s * AGG_ROWS_PER_TILE
    pltpu.sync_copy(agg_sh.at[pl.ds(aggbase, AGG_ROWS_PER_TILE)],
                    out_hbm.at[pl.ds(outbase, AGG_ROWS_PER_TILE)])


_SC_AGG_CACHE = []


def _sc_agg(m_rows, src2d, dst2d, ew):
    if not _SC_AGG_CACHE:
        _SC_AGG_CACHE.append(_build_sc_agg())
    return _SC_AGG_CACHE[0](m_rows, src2d, dst2d, ew)


def _build_sc_agg():
    return pl.kernel(
        _sc_agg_body,
        out_type=jax.ShapeDtypeStruct((2 * NP, F), jnp.float32),
        mesh=plsc.VectorSubcoreMesh(core_axis_name="c", subcore_axis_name="s"),
        compiler_params=pltpu.CompilerParams(use_tc_tiling_on_sc=False),
        scratch_types=[
        pltpu.VMEM_SHARED((NP, F), jnp.float32),
        pltpu.VMEM((CHUNK, F), jnp.float32),
        pltpu.VMEM((NCHUNK, CHUNK), jnp.int32),
        pltpu.VMEM((NCHUNK, CHUNK), jnp.int32),
        pltpu.VMEM((SUPER,), jnp.float32),
        pltpu.VMEM((SUPER, F), jnp.float32),
        pltpu.SemaphoreType.DMA,
        ],
    )


def _dot(a, b):
    return lax.dot_general(a, b, (((1,), (0,)), ((), ())),
                           preferred_element_type=jnp.float32)


def _sigmoid(x):
    return 1.0 / (1.0 + jnp.exp(-x))


def _make_gru_body(relu):
    def body(h_ref, agg_ref, wn_ref, wir_ref, wiz_ref, win_ref,
             whr_ref, whz_ref, whn_ref, b_ref, hout_ref, mout_ref):
        h = h_ref[...]
        agg = agg_ref[0] + agg_ref[1]
        i_r = _dot(agg, wir_ref[...]) + b_ref[0:1, :]
        i_z = _dot(agg, wiz_ref[...]) + b_ref[1:2, :]
        i_n = _dot(agg, win_ref[...]) + b_ref[2:3, :]
        h_r = _dot(h, whr_ref[...]) + b_ref[3:4, :]
        h_z = _dot(h, whz_ref[...]) + b_ref[4:5, :]
        h_n = _dot(h, whn_ref[...]) + b_ref[5:6, :]
        r = _sigmoid(i_r + h_r)
        z = _sigmoid(i_z + h_z)
        n = jnp.tanh(i_n + r * h_n)
        hn = (1.0 - z) * n + z * h
        if relu:
            hn = jnp.maximum(hn, 0.0)
        hout_ref[...] = hn
        mout_ref[...] = _dot(hn, wn_ref[...])
    return body


def _tc_gru(h_p, agg3, wn, wir, wiz, win, whr, whz, whn, b6, relu):
    wspec = pl.BlockSpec((128, 128), lambda i: (0, 0))
    return pl.pallas_call(
        _make_gru_body(relu),
        grid=(NPACK // GBLK,),
        in_specs=[
            pl.BlockSpec((GBLK, 128), lambda i: (i, 0)),
            pl.BlockSpec((2, GBLK, 128), lambda i: (0, i, 0)),
            wspec, wspec, wspec, wspec, wspec, wspec, wspec,
            pl.BlockSpec((6, 128), lambda i: (0, 0)),
        ],
        out_specs=[pl.BlockSpec((GBLK, 128), lambda i: (i, 0))] * 2,
        out_shape=[jax.ShapeDtypeStruct((NPACK, 128), jnp.float32)] * 2,
    )(h_p, agg3, wn, wir, wiz, win, whr, whz, whn, b6)


def _mm_body(x_ref, w_ref, o_ref):
    o_ref[...] = _dot(x_ref[...], w_ref[...])


def _tc_matmul(x_p, wbd):
    return pl.pallas_call(
        _mm_body,
        grid=(NPACK // GBLK,),
        in_specs=[
            pl.BlockSpec((GBLK, 128), lambda i: (i, 0)),
            pl.BlockSpec((128, 128), lambda i: (0, 0)),
        ],
        out_specs=pl.BlockSpec((GBLK, 128), lambda i: (i, 0)),
        out_shape=jax.ShapeDtypeStruct((NPACK, 128), jnp.float32),
    )(x_p, wbd)


def _ls_body(h_ref, o_ref):
    v = h_ref[...]
    mx = jnp.max(v, axis=1, keepdims=True)
    ex = jnp.exp(v - mx)
    sm = jnp.sum(ex, axis=1, keepdims=True)
    o_ref[...] = v - mx - jnp.log(sm)


def _tc_logsoftmax(h2):
    return pl.pallas_call(
        _ls_body,
        grid=(10,),
        in_specs=[pl.BlockSpec((N // 10, F), lambda i: (i, 0))],
        out_specs=pl.BlockSpec((N // 10, F), lambda i: (i, 0)),
        out_shape=jax.ShapeDtypeStruct((N, F), jnp.float32),
    )(h2)


def kernel(x, edge_index, edge_weight, weight1, w_ih1, w_hh1, b_ih1, b_hh1,
           weight2, w_ih2, w_hh2, b_ih2, b_hh2):
    f32 = jnp.float32

    xp = jnp.pad(x, ((0, NP - N), (0, 0)))
    pad_e = EP - E
    src = jnp.concatenate([edge_index[0].astype(jnp.int32),
                           jnp.zeros((pad_e,), jnp.int32)])
    dst = jnp.concatenate([edge_index[1].astype(jnp.int32),
                           jnp.zeros((pad_e,), jnp.int32)])
    ew = jnp.concatenate([edge_weight.astype(f32), jnp.zeros((pad_e,), f32)])
    src2d = src.reshape(IDXROWS, CHUNK)
    dst2d = dst.reshape(IDXROWS, CHUNK)

    eye8 = jnp.eye(PACK, dtype=f32)

    def bd(w16):
        return jnp.kron(eye8, w16.astype(f32))

    def biases6(b_ih, b_hh):
        parts = [b_ih[0:16], b_ih[16:32], b_ih[32:48],
                 b_hh[0:16], b_hh[16:32], b_hh[32:48]]
        return jnp.stack([jnp.tile(p.astype(f32), (PACK,)) for p in parts])

    def gate_w(w_ih, w_hh):
        return (bd(w_ih[0:16].T), bd(w_ih[16:32].T), bd(w_ih[32:48].T),
                bd(w_hh[0:16].T), bd(w_hh[16:32].T), bd(w_hh[32:48].T))

    g1 = gate_w(w_ih1, w_hh1)
    g2 = gate_w(w_ih2, w_hh2)
    b61 = biases6(b_ih1, b_hh1)
    b62 = biases6(b_ih2, b_hh2)

    wn_list = [bd(weight1[i]) for i in range(1, 16)]
    wn_list.append(bd(weight2[0]))
    wn_list.append(bd(weight2[1]))
    wn_list.append(bd(jnp.eye(F, dtype=f32)))

    h_p = xp.reshape(NPACK, 128)
    m_p = _tc_matmul(h_p, bd(weight1[0]))

    for i in range(18):
        m_rows = m_p.reshape(NP, F)
        aggp = _sc_agg(m_rows, src2d, dst2d, ew)
        agg3 = aggp.reshape(2, NPACK, 128)
        if i < 16:
            gates, b6 = g1, b61
        else:
            gates, b6 = g2, b62
        h_p, m_p = _tc_gru(h_p, agg3, wn_list[i], *gates, b6,
                           relu=(i == 15))

    h2 = h_p.reshape(NP, F)[:N]
    return _tc_logsoftmax(h2)
